# Initial kernel scaffold; baseline (speedup 1.0000x reference)
import jax
import jax.numpy as jnp
from jax.experimental import pallas as pl


def kernel(image, eps, enc_head_w, enc_head_b, enc_blk_w, enc_blk_b, mu_w, mu_b, lv_w, lv_b, gate_w, gate_b, exp_w1, exp_b1, exp_w2, exp_b2, dec_out_w, dec_out_b, fus_head_w, fus_head_b, fus_blk_w, fus_blk_b, fus_out_w, fus_out_b):
    raise NotImplementedError("write your pallas kernel here")


# speedup vs baseline: 1.1442x; 1.1442x over previous
import functools

import jax
import jax.numpy as jnp
from jax.experimental import pallas as pl

_B, _H, _W = 2, 224, 224
_TH = 28
_NT = _H // _TH
_ALPHA = 1.0


def _overlap(x):
    xp = jnp.pad(x, ((0, 0), (1, 1), (1, 1), (0, 0)))
    tiles = [xp[:, i * _TH:i * _TH + _TH + 2] for i in range(_NT)]
    return jnp.stack(tiles, axis=1)


def _acc_conv(x, w_ref, cin, pw):
    acc = None
    for dy in range(3):
        for dx in range(3):
            xs = x[dy:dy + _TH, dx:dx + _W, :].reshape(_TH * _W, cin)
            wt = w_ref[0, dy, dx] if pw else w_ref[dy, dx]
            p = jax.lax.dot_general(xs, wt, (((1,), (0,)), ((), ())),
                                    preferred_element_type=jnp.float32)
            acc = p if acc is None else acc + p
    return acc


def _conv_body(x_ref, w_ref, b_ref, o_ref, *, act, cin, cout, pw):
    acc = _acc_conv(x_ref[0, 0], w_ref, cin, pw) + b_ref[0]
    if act:
        acc = jnp.maximum(acc, 0.0)
    o_ref[...] = acc.reshape(1, _TH, _W, cout)


def _conv3x3(x, w, b, act=False):
    cin = x.shape[-1]
    pw = (w.ndim == 5)
    cout = w.shape[-1]
    xo = _overlap(x)
    if b.ndim == 1:
        b = jnp.broadcast_to(b[None], (_B, cout))
    b = b.reshape(_B, 1, cout)
    if pw:
        wspec = pl.BlockSpec((1, 3, 3, cin, cout), lambda bb, i: (bb, 0, 0, 0, 0))
    else:
        wspec = pl.BlockSpec((3, 3, cin, cout), lambda bb, i: (0, 0, 0, 0))
    return pl.pallas_call(
        functools.partial(_conv_body, act=act, cin=cin, cout=cout, pw=pw),
        grid=(_B, _NT),
        in_specs=[
            pl.BlockSpec((1, 1, _TH + 2, _W + 2, cin), lambda bb, i: (bb, i, 0, 0, 0)),
            wspec,
            pl.BlockSpec((1, 1, cout), lambda bb, i: (bb, 0, 0)),
        ],
        out_specs=pl.BlockSpec((1, _TH, _W, cout), lambda bb, i: (bb, i, 0, 0)),
        out_shape=jax.ShapeDtypeStruct((_B, _H, _W, cout), jnp.float32),
    )(xo, w, b)


def _viz_body(x_ref, w_ref, mb_ref, lb_ref, eps_ref,
              mu_ref, s2_ref, z_ref, ps_ref):
    acc = _acc_conv(x_ref[0, 0], w_ref, 64, False)
    mu = acc[:, :64] + mb_ref[0]
    lv = acc[:, 64:] + lb_ref[0]
    s2 = jnp.exp(lv)
    z = mu + _ALPHA * jnp.exp(0.5 * lv) * eps_ref[...].reshape(_TH * _W, 64)
    mu_ref[...] = mu.reshape(1, _TH, _W, 64)
    s2_ref[...] = s2.reshape(1, _TH, _W, 64)
    z_ref[...] = z.reshape(1, _TH, _W, 64)

    @pl.when(pl.program_id(1) == 0)
    def _():
        ps_ref[...] = jnp.zeros_like(ps_ref)
    ps_ref[...] += jnp.sum(z, axis=0, keepdims=True)[None]


def _viz_call(h, wcat, mu_b, lv_b, eps):
    f32 = jnp.float32
    return pl.pallas_call(
        _viz_body,
        grid=(_B, _NT),
        in_specs=[
            pl.BlockSpec((1, 1, _TH + 2, _W + 2, 64), lambda bb, i: (bb, i, 0, 0, 0)),
            pl.BlockSpec((3, 3, 64, 128), lambda bb, i: (0, 0, 0, 0)),
            pl.BlockSpec((1, 64), lambda bb, i: (0, 0)),
            pl.BlockSpec((1, 64), lambda bb, i: (0, 0)),
            pl.BlockSpec((1, _TH, _W, 64), lambda bb, i: (bb, i, 0, 0)),
        ],
        out_specs=[
            pl.BlockSpec((1, _TH, _W, 64), lambda bb, i: (bb, i, 0, 0)),
            pl.BlockSpec((1, _TH, _W, 64), lambda bb, i: (bb, i, 0, 0)),
            pl.BlockSpec((1, _TH, _W, 64), lambda bb, i: (bb, i, 0, 0)),
            pl.BlockSpec((1, 1, 64), lambda bb, i: (bb, 0, 0)),
        ],
        out_shape=[
            jax.ShapeDtypeStruct((_B, _H, _W, 64), f32),
            jax.ShapeDtypeStruct((_B, _H, _W, 64), f32),
            jax.ShapeDtypeStruct((_B, _H, _W, 64), f32),
            jax.ShapeDtypeStruct((_B, 1, 64), f32),
        ],
    )(_overlap(h), wcat, mu_b[None], lv_b[None], eps)


def _dec_body(x_ref, w_ref, b_ref, img_ref, n_ref, ic_ref):
    n = _acc_conv(x_ref[0, 0], w_ref, 64, False) + b_ref[0]
    img = img_ref[...].reshape(_TH * _W, 1)
    n_ref[...] = n.reshape(1, _TH, _W, 1)
    ic_ref[...] = (img - n).reshape(1, _TH, _W, 1)


def _dec_call(combined, w, b, image_nhwc):
    f32 = jnp.float32
    return pl.pallas_call(
        _dec_body,
        grid=(_B, _NT),
        in_specs=[
            pl.BlockSpec((1, 1, _TH + 2, _W + 2, 64), lambda bb, i: (bb, i, 0, 0, 0)),
            pl.BlockSpec((3, 3, 64, 1), lambda bb, i: (0, 0, 0, 0)),
            pl.BlockSpec((1, 1), lambda bb, i: (0, 0)),
            pl.BlockSpec((1, _TH, _W, 1), lambda bb, i: (bb, i, 0, 0)),
        ],
        out_specs=[
            pl.BlockSpec((1, _TH, _W, 1), lambda bb, i: (bb, i, 0, 0)),
            pl.BlockSpec((1, _TH, _W, 1), lambda bb, i: (bb, i, 0, 0)),
        ],
        out_shape=[
            jax.ShapeDtypeStruct((_B, _H, _W, 1), f32),
            jax.ShapeDtypeStruct((_B, _H, _W, 1), f32),
        ],
    )(_overlap(combined), w, b[None], image_nhwc)


def _hwio(w):
    return jnp.transpose(w, (2, 3, 1, 0))


def kernel(image, eps, enc_head_w, enc_head_b, enc_blk_w, enc_blk_b,
           mu_w, mu_b, lv_w, lv_b, gate_w, gate_b,
           exp_w1, exp_b1, exp_w2, exp_b2, dec_out_w, dec_out_b,
           fus_head_w, fus_head_b, fus_blk_w, fus_blk_b, fus_out_w, fus_out_b):
    image_nhwc = jnp.transpose(image, (0, 2, 3, 1))
    eps_nhwc = jnp.transpose(eps, (0, 2, 3, 1))

    h = _conv3x3(image_nhwc, _hwio(enc_head_w), enc_head_b, act=True)
    for i in range(enc_blk_w.shape[0]):
        h = _conv3x3(h, _hwio(enc_blk_w[i]), enc_blk_b[i], act=True)

    wcat = jnp.concatenate([_hwio(mu_w), _hwio(lv_w)], axis=-1)
    mu, sigma2, z, psum = _viz_call(h, wcat, mu_b, lv_b, eps_nhwc)

    pooled = psum.reshape(_B, 64) / float(_H * _W)
    logits = pooled @ gate_w + gate_b
    topv, topi = jax.lax.top_k(logits, 2)
    probs = jax.nn.softmax(topv, axis=-1)

    w1h = jnp.transpose(exp_w1, (0, 3, 4, 2, 1))
    w2h = jnp.transpose(exp_w2, (0, 3, 4, 2, 1))
    w1cat = jnp.transpose(w1h[topi], (0, 2, 3, 4, 1, 5)).reshape(_B, 3, 3, 64, 128)
    b1cat = exp_b1[topi].reshape(_B, 128)
    w2sel = w2h[topi] * probs[:, :, None, None, None, None]
    w2cat = jnp.transpose(w2sel, (0, 2, 3, 1, 4, 5)).reshape(_B, 3, 3, 128, 64)
    b2c = jnp.einsum('bk,bko->bo', probs, exp_b2[topi])

    h1 = _conv3x3(z, w1cat, b1cat, act=True)
    combined = _conv3x3(h1, w2cat, b2c)

    n_nhwc, ic = _dec_call(combined, _hwio(dec_out_w), dec_out_b, image_nhwc)

    f = _conv3x3(ic, _hwio(fus_head_w), fus_head_b, act=True)
    for i in range(fus_blk_w.shape[0]):
        f = _conv3x3(f, _hwio(fus_blk_w[i]), fus_blk_b[i], act=True)
    out_nhwc = _conv3x3(f, _hwio(fus_out_w), fus_out_b)

    out = jnp.transpose(out_nhwc, (0, 3, 1, 2))
    n = jnp.transpose(n_nhwc, (0, 3, 1, 2))
    mu_o = jnp.transpose(mu, (0, 3, 1, 2))
    s2_o = jnp.transpose(sigma2, (0, 3, 1, 2))
    return (out, n, mu_o, s2_o)
